# Initial kernel scaffold; baseline (speedup 1.0000x reference)
import jax
import jax.numpy as jnp
from jax.experimental import pallas as pl


def kernel(x, token_table, pos_table):
    raise NotImplementedError("write your pallas kernel here")


# speedup vs baseline: 3.3085x; 3.3085x over previous
import functools

import jax
import jax.numpy as jnp
from jax import lax
from jax.experimental import pallas as pl
from jax.experimental.pallas import tpu as pltpu
from jax.experimental.pallas import tpu_sc as plsc

VOCAB = 100000
EMBED_DIM = 64
BATCH = 4096
SEQ_LEN = 200

NUM_CORES = 2
NUM_SUBCORES = 16
NUM_WORKERS = NUM_CORES * NUM_SUBCORES

FLAT = BATCH * SEQ_LEN
PER_W = FLAT // NUM_WORKERS
SEQS_PER_CHUNK = 4
CHUNK = SEQS_PER_CHUNK * SEQ_LEN
N_CHUNKS = PER_W // CHUNK


def _embed_body(x_hbm, tok_hbm, pos_hbm, out_hbm, idx_v, rows_v, sem):
  cid = lax.axis_index("c")
  sid = lax.axis_index("s")
  wid = sid * NUM_CORES + cid
  base = wid * PER_W

  @pl.loop(0, N_CHUNKS)
  def _chunk(ci):
    off = base + ci * CHUNK
    pltpu.sync_copy(x_hbm.at[pl.ds(off, CHUNK)], idx_v)
    pltpu.sync_copy(pos_hbm, rows_v)
    pltpu.async_copy(tok_hbm.at[idx_v], rows_v, sem, add=True).wait()
    pltpu.sync_copy(rows_v, out_hbm.at[pl.ds(off, CHUNK)])


@jax.jit
def _embed(x_flat, token_table, pos_table):
  mesh = plsc.VectorSubcoreMesh(
      core_axis_name="c", subcore_axis_name="s",
      num_cores=NUM_CORES, num_subcores=NUM_SUBCORES,
  )
  run = pl.kernel(
      _embed_body,
      out_type=jax.ShapeDtypeStruct((FLAT, EMBED_DIM), jnp.float32),
      mesh=mesh,
      compiler_params=pltpu.CompilerParams(use_tc_tiling_on_sc=False),
      scratch_types=[
          pltpu.VMEM((CHUNK,), jnp.int32),
          pltpu.VMEM((CHUNK, EMBED_DIM), jnp.float32),
          pltpu.SemaphoreType.DMA,
      ],
  )
  return run(x_flat, token_table, pos_table)


def kernel(x, token_table, pos_table):
  x_flat = x.reshape(FLAT).astype(jnp.int32)
  pos_block = jnp.tile(pos_table, (SEQS_PER_CHUNK, 1))
  out = _embed(x_flat, token_table, pos_block)
  return out.reshape(BATCH, SEQ_LEN, EMBED_DIM)
